# baseline (speedup 1.0000x reference)
import functools

import jax
import jax.numpy as jnp
from jax import lax
from jax.experimental import pallas as pl
from jax.experimental.pallas import tpu as pltpu
from jax.experimental.pallas import tpu_sc as plsc


def _make_sc_kernel(batch, dim):
    info = plsc.get_sparse_core_info()
    nc, ns, lanes = info.num_cores, info.num_subcores, info.num_lanes
    nw = nc * ns
    bpw = batch // nw
    assert batch % nw == 0 and dim % lanes == 0
    nchunk = dim // lanes

    mesh = plsc.VectorSubcoreMesh(core_axis_name="c", subcore_axis_name="s")

    @functools.partial(
        pl.kernel,
        mesh=mesh,
        out_type=jax.ShapeDtypeStruct((batch, dim * dim), jnp.float32),
        scratch_types=[
            pltpu.VMEM((bpw, dim), jnp.float32),
            pltpu.VMEM((dim * dim,), jnp.float32),
            pltpu.VMEM((dim * dim,), jnp.float32),
            pltpu.SemaphoreType.DMA,
            pltpu.SemaphoreType.DMA,
        ],
        compiler_params=pltpu.CompilerParams(needs_layout_passes=False),
    )
    def diag_sc(e_hbm, z_hbm, out_hbm, ebuf, mat0, mat1, sem0, sem1):
        wid = lax.axis_index("s") * nc + lax.axis_index("c")
        base = wid * bpw
        pltpu.sync_copy(e_hbm.at[pl.ds(base, bpw)], ebuf)
        pltpu.sync_copy(z_hbm, mat0)
        pltpu.sync_copy(z_hbm, mat1)
        mats = (mat0, mat1)
        sems = (sem0, sem1)
        handles = [None, None]
        for b in range(bpw):
            mat = mats[b % 2]
            if handles[b % 2] is not None:
                handles[b % 2].wait()
            for k in range(nchunk):
                idx = (jnp.arange(lanes, dtype=jnp.int32) + (k * lanes)) * (dim + 1)
                vals = jnp.exp(ebuf[b, pl.ds(k * lanes, lanes)])
                plsc.store_scatter(mat, [idx], vals)
            handles[b % 2] = pltpu.async_copy(mat, out_hbm.at[base + b],
                                              sems[b % 2])
        handles[0].wait()
        handles[1].wait()

    return diag_sc


def kernel(embeddings):
    batch, dim, _ = embeddings.shape
    e2 = embeddings[:, :, 0]
    zeros = jnp.zeros((dim * dim,), dtype=jnp.float32)
    sc = _make_sc_kernel(batch, dim)
    return sc(e2, zeros).reshape(batch, dim, dim)


# speedup vs baseline: 1.7988x; 1.7988x over previous
import functools

import jax
import jax.numpy as jnp
from jax import lax
from jax.experimental import pallas as pl
from jax.experimental.pallas import tpu as pltpu
from jax.experimental.pallas import tpu_sc as plsc

_CH = 2


def _make_sc_kernel(batch, dim):
    info = plsc.get_sparse_core_info()
    nc, ns, lanes = info.num_cores, info.num_subcores, info.num_lanes
    nw = nc * ns
    bpw = batch // nw
    assert batch % nw == 0 and dim % lanes == 0 and bpw % _CH == 0
    nchunk = dim // lanes
    msize = dim * dim

    mesh = plsc.VectorSubcoreMesh(core_axis_name="c", subcore_axis_name="s")

    @functools.partial(
        pl.kernel,
        mesh=mesh,
        out_type=jax.ShapeDtypeStruct((batch * msize,), jnp.float32),
        scratch_types=[
            pltpu.VMEM((bpw, dim), jnp.float32),
            pltpu.VMEM((_CH * msize,), jnp.float32),
            pltpu.VMEM((_CH * msize,), jnp.float32),
            pltpu.SemaphoreType.DMA,
            pltpu.SemaphoreType.DMA,
        ],
        compiler_params=pltpu.CompilerParams(needs_layout_passes=False),
    )
    def diag_sc(e_hbm, z_hbm, out_hbm, ebuf, buf0, buf1, sem0, sem1):
        wid = lax.axis_index("s") * nc + lax.axis_index("c")
        base = wid * bpw
        pltpu.sync_copy(e_hbm.at[pl.ds(base, bpw)], ebuf)
        pltpu.sync_copy(z_hbm, buf0)
        pltpu.sync_copy(z_hbm, buf1)
        bufs = (buf0, buf1)
        sems = (sem0, sem1)
        handles = [None, None]
        for c in range(bpw // _CH):
            buf = bufs[c % 2]
            if handles[c % 2] is not None:
                handles[c % 2].wait()
            for m in range(_CH):
                b = c * _CH + m
                for k in range(nchunk):
                    idx = (jnp.arange(lanes, dtype=jnp.int32)
                           + (k * lanes)) * (dim + 1) + (m * msize)
                    vals = jnp.exp(ebuf[b, pl.ds(k * lanes, lanes)])
                    plsc.store_scatter(buf, [idx], vals)
            dst = out_hbm.at[pl.ds((base + c * _CH) * msize, _CH * msize)]
            handles[c % 2] = pltpu.async_copy(buf, dst, sems[c % 2])
        handles[0].wait()
        handles[1].wait()

    return diag_sc


def kernel(embeddings):
    batch, dim, _ = embeddings.shape
    e2 = embeddings[:, :, 0]
    zeros = jnp.zeros((_CH * dim * dim,), dtype=jnp.float32)
    sc = _make_sc_kernel(batch, dim)
    return sc(e2, zeros).reshape(batch, dim, dim)
